# Initial kernel scaffold; baseline (speedup 1.0000x reference)
import jax
import jax.numpy as jnp
from jax.experimental import pallas as pl


def kernel(x, edge_attr, edge_distance, params, edge_index):
    raise NotImplementedError("write your pallas kernel here")


# speedup vs baseline: 2.5244x; 2.5244x over previous
import functools
import math

import jax
import jax.numpy as jnp
from jax import lax
from jax.experimental import pallas as pl
from jax.experimental.pallas import tpu as pltpu
from jax.experimental.pallas import tpu_sc as plsc

_F32 = jnp.float32
_HIGH = jax.lax.Precision.HIGHEST


def _hdot(a, b):
    return jnp.dot(a, b, precision=_HIGH)


def _tc_matmul(xx, wt):
    M, K = xx.shape
    Dout = wt.shape[1]
    BM = 1024
    grid = (pl.cdiv(M, BM),)

    def body(x_ref, w_ref, o_ref):
        o_ref[...] = jnp.dot(x_ref[...], w_ref[...], precision=_HIGH,
                             preferred_element_type=_F32)

    return pl.pallas_call(
        body,
        grid=grid,
        in_specs=[pl.BlockSpec((BM, K), lambda i: (i, 0)),
                  pl.BlockSpec((K, Dout), lambda i: (0, 0))],
        out_specs=pl.BlockSpec((BM, Dout), lambda i: (i, 0)),
        out_shape=jax.ShapeDtypeStruct((M, Dout), _F32),
    )(xx, wt)


def _tc_edge(g0, ea, ed, u0r, v0r, w0r, c0r):
    EP, D = g0.shape
    BM = 2048
    grid = (pl.cdiv(EP, BM),)

    def body(g_ref, ea_ref, ed_ref, u_ref, v_ref, w_ref, c_ref,
             mh_ref, mt_ref):
        ang = ea_ref[...] * (math.pi / 180.0)
        s = jnp.sin(ang)
        co = jnp.cos(ang)
        d = ed_ref[...]
        t = (g_ref[...] + s * u_ref[...] + co * v_ref[...]
             + d * w_ref[...] + c_ref[...])
        mh_ref[...] = 0.5 * t * (1.0 + lax.erf(t * (1.0 / math.sqrt(2.0))))
        col = lax.broadcasted_iota(jnp.int32, (BM, D), 1)
        mt = jnp.where(col == 0, s, jnp.zeros((BM, D), _F32))
        mt = jnp.where(col == 1, co, mt)
        mt = jnp.where(col == 2, d, mt)
        mt_ref[...] = jnp.where(col == 3, jnp.ones((BM, D), _F32), mt)

    row = pl.BlockSpec((1, D), lambda i: (0, 0))
    blk = pl.BlockSpec((BM, D), lambda i: (i, 0))
    return pl.pallas_call(
        body,
        grid=grid,
        in_specs=[blk,
                  pl.BlockSpec((BM, 1), lambda i: (i, 0)),
                  pl.BlockSpec((BM, 1), lambda i: (i, 0)),
                  row, row, row, row],
        out_specs=[blk, blk],
        out_shape=[jax.ShapeDtypeStruct((EP, D), _F32)] * 2,
    )(g0, ea, ed, u0r, v0r, w0r, c0r)


def _tc_combine(Ph0, Ph1, Pt0, Pt1, wa0t, b1t, uh, vh, wh, ch, cb0r, cb1r):
    NPAD, D = Ph0.shape
    BM = 2048
    grid = (pl.cdiv(NPAD, BM),)

    def body(ph0_ref, ph1_ref, pt0_ref, pt1_ref, wa_ref, b1_ref,
             u_ref, v_ref, w_ref, c_ref, k0_ref, k1_ref,
             z_ref, r_ref, cinv_ref):
        S = ph0_ref[...] + ph1_ref[...]
        T = pt0_ref[...] + pt1_ref[...]
        col = lax.broadcasted_iota(jnp.int32, (BM, D), 1)

        def getcol(j):
            return jnp.sum(jnp.where(col == j, T, jnp.zeros((BM, D), _F32)),
                           axis=1, keepdims=True)

        cnt = getcol(3)
        inv = 1.0 / jnp.maximum(cnt, 1.0)
        mask = (cnt > 0.0).astype(_F32)
        aggr0 = S * inv
        h1 = jnp.tanh(jnp.dot(aggr0, wa_ref[...], precision=_HIGH,
                              preferred_element_type=_F32) + k0_ref[...])
        z_ref[...] = jnp.dot(h1, b1_ref[...], precision=_HIGH,
                             preferred_element_type=_F32)
        r_ref[...] = ((getcol(0) * inv) * u_ref[...]
                      + (getcol(1) * inv) * v_ref[...]
                      + (getcol(2) * inv) * w_ref[...]
                      + mask * c_ref[...] + k1_ref[...])
        cinv_ref[...] = jnp.broadcast_to(inv, (BM, D))

    row = pl.BlockSpec((1, D), lambda i: (0, 0))
    blk = pl.BlockSpec((BM, D), lambda i: (i, 0))
    outs = [jax.ShapeDtypeStruct((NPAD, D), _F32)] * 3
    return pl.pallas_call(
        body,
        grid=grid,
        in_specs=[blk, blk, blk, blk,
                  pl.BlockSpec((D, D), lambda i: (0, 0)),
                  pl.BlockSpec((D, D), lambda i: (0, 0)),
                  row, row, row, row, row, row],
        out_specs=[blk] * 3,
        out_shape=outs,
    )(Ph0, Ph1, Pt0, Pt1, wa0t, b1t, uh, vh, wh, ch, cb0r, cb1r)


def _tc_final(Q0, Q1, C, R):
    NPAD, D = Q0.shape
    BM = 2504
    grid = (pl.cdiv(NPAD, BM),)

    def body(q0_ref, q1_ref, c_ref, r_ref, o_ref):
        o_ref[...] = (q0_ref[...] + q1_ref[...]) * c_ref[...] + r_ref[...]

    spec = pl.BlockSpec((BM, D), lambda i: (i, 0))
    return pl.pallas_call(
        body,
        grid=grid,
        in_specs=[spec] * 4,
        out_specs=spec,
        out_shape=jax.ShapeDtypeStruct((NPAD, D), _F32),
    )(Q0, Q1, C, R)


_CH = 128


def _sc_gather(table, idx):
    Nt, D = table.shape
    EP = idx.shape[0]
    info = plsc.get_sparse_core_info()
    NC, NS = info.num_cores, info.num_subcores
    NW = NC * NS
    per_w = EP // NW
    n_ch = per_w // _CH
    mesh = plsc.VectorSubcoreMesh(core_axis_name="c", subcore_axis_name="s")

    @functools.partial(
        pl.kernel, mesh=mesh,
        out_type=jax.ShapeDtypeStruct((EP, D), _F32),
        scratch_types=[pltpu.VMEM((_CH,), jnp.int32),
                       pltpu.VMEM((_CH, D), _F32),
                       pltpu.SemaphoreType.DMA],
    )
    def k(table_hbm, idx_hbm, out_hbm, idx_v, rows_v, sem):
        wid = lax.axis_index("s") * NC + lax.axis_index("c")
        base = wid * per_w

        def step(i, carry):
            off = base + i * _CH
            pltpu.sync_copy(idx_hbm.at[pl.ds(off, _CH)], idx_v)
            pltpu.async_copy(table_hbm.at[idx_v], rows_v, sem).wait()
            pltpu.sync_copy(rows_v, out_hbm.at[pl.ds(off, _CH)])
            return carry

        lax.fori_loop(0, n_ch, step, 0)

    return k(table, idx)


def _sc_scatter_add(msgs, dst, npad):
    EP, W = msgs.shape
    info = plsc.get_sparse_core_info()
    NC, NS = info.num_cores, info.num_subcores
    NW = NC * NS
    per_w = EP // NW
    n_ch = per_w // _CH
    tile_rows = npad // NS
    mesh = plsc.VectorSubcoreMesh(core_axis_name="c", subcore_axis_name="s")
    zeros = jnp.zeros((npad, W), _F32)

    @functools.partial(
        pl.kernel, mesh=mesh,
        out_type=jax.ShapeDtypeStruct((NC * npad, W), _F32),
        scratch_types=[pltpu.VMEM((_CH,), jnp.int32),
                       pltpu.VMEM((_CH, W), _F32),
                       pltpu.VMEM_SHARED((npad, W), _F32),
                       pltpu.SemaphoreType.DMA],
    )
    def k(msgs_hbm, dst_hbm, zeros_hbm, out_hbm, idx_v, rows_v, acc, sem):
        cid = lax.axis_index("c")
        sid = lax.axis_index("s")
        wid = sid * NC + cid
        trow = sid * tile_rows
        pltpu.sync_copy(zeros_hbm.at[pl.ds(trow, tile_rows)],
                        acc.at[pl.ds(trow, tile_rows)])
        plsc.subcore_barrier()
        base = wid * per_w

        def step(i, carry):
            off = base + i * _CH
            pltpu.sync_copy(dst_hbm.at[pl.ds(off, _CH)], idx_v)
            pltpu.sync_copy(msgs_hbm.at[pl.ds(off, _CH)], rows_v)
            pltpu.sync_copy(rows_v, acc.at[idx_v], add=True)
            return carry

        lax.fori_loop(0, n_ch, step, 0)
        plsc.subcore_barrier()
        pltpu.sync_copy(acc.at[pl.ds(trow, tile_rows)],
                        out_hbm.at[pl.ds(cid * npad + trow, tile_rows)])

    return k(msgs, dst, zeros)


def _sc_gather_scatter(table, src, dst, npad):
    Nt, D = table.shape
    EP = src.shape[0]
    info = plsc.get_sparse_core_info()
    NC, NS = info.num_cores, info.num_subcores
    NW = NC * NS
    per_w = EP // NW
    n_ch = per_w // _CH
    tile_rows = npad // NS
    mesh = plsc.VectorSubcoreMesh(core_axis_name="c", subcore_axis_name="s")
    zeros = jnp.zeros((npad, D), _F32)

    @functools.partial(
        pl.kernel, mesh=mesh,
        out_type=jax.ShapeDtypeStruct((NC * npad, D), _F32),
        scratch_types=[pltpu.VMEM((_CH,), jnp.int32),
                       pltpu.VMEM((_CH,), jnp.int32),
                       pltpu.VMEM((_CH, D), _F32),
                       pltpu.VMEM_SHARED((npad, D), _F32),
                       pltpu.SemaphoreType.DMA],
    )
    def k(table_hbm, src_hbm, dst_hbm, zeros_hbm, out_hbm,
          sidx_v, didx_v, rows_v, acc, sem):
        cid = lax.axis_index("c")
        sid = lax.axis_index("s")
        wid = sid * NC + cid
        trow = sid * tile_rows
        pltpu.sync_copy(zeros_hbm.at[pl.ds(trow, tile_rows)],
                        acc.at[pl.ds(trow, tile_rows)])
        plsc.subcore_barrier()
        base = wid * per_w

        def step(i, carry):
            off = base + i * _CH
            pltpu.sync_copy(src_hbm.at[pl.ds(off, _CH)], sidx_v)
            pltpu.sync_copy(dst_hbm.at[pl.ds(off, _CH)], didx_v)
            pltpu.async_copy(table_hbm.at[sidx_v], rows_v, sem).wait()
            pltpu.sync_copy(rows_v, acc.at[didx_v], add=True)
            return carry

        lax.fori_loop(0, n_ch, step, 0)
        plsc.subcore_barrier()
        pltpu.sync_copy(acc.at[pl.ds(trow, tile_rows)],
                        out_hbm.at[pl.ds(cid * npad + trow, tile_rows)])

    return k(table, src, dst, zeros)


def _compose_affine(p, D):
    inf = D + 3
    M = jnp.eye(inf, dtype=_F32)
    c = jnp.zeros((inf,), _F32)
    for (v, g, b) in p["wn"]:
        Wn = g * v / jnp.linalg.norm(v, axis=1, keepdims=True)
        M = _hdot(Wn, M)
        c = _hdot(Wn, c) + b
    return _hdot(p["Wf"], M), _hdot(p["Wf"], c) + p["bf"]


def kernel(x, edge_attr, edge_distance, params, edge_index):
    N, D = x.shape
    E = edge_index.shape[1]
    src = edge_index[0].astype(jnp.int32)
    dst = edge_index[1].astype(jnp.int32)

    EP = ((E + 4095) // 4096) * 4096
    NPAD = ((N + 1 + 127) // 128) * 128
    pad = EP - E
    src_p = jnp.concatenate([src, jnp.zeros((pad,), jnp.int32)])
    dst_p = jnp.concatenate([dst, jnp.full((pad,), N, jnp.int32)])
    ea_p = jnp.concatenate([edge_attr.astype(_F32),
                            jnp.zeros((pad,), _F32)]).reshape(EP, 1)
    ed_p = jnp.concatenate([edge_distance.astype(_F32),
                            jnp.zeros((pad,), _F32)]).reshape(EP, 1)

    p0, p1 = params["layers"]
    Mf0, cf0 = _compose_affine(p0, D)
    u0r = Mf0[:, D].reshape(1, D)
    v0r = Mf0[:, D + 1].reshape(1, D)
    w0r = Mf0[:, D + 2].reshape(1, D)
    c0r = cf0.reshape(1, D)
    wa0t = p0["Wa"].T
    cb0r = (p0["ba"] + p0["bias"]).reshape(1, D)

    Mf1, cf1 = _compose_affine(p1, D)
    b1t = _hdot(p1["Wa"], Mf1[:, :D]).T
    uh = _hdot(p1["Wa"], Mf1[:, D]).reshape(1, D)
    vh = _hdot(p1["Wa"], Mf1[:, D + 1]).reshape(1, D)
    wh = _hdot(p1["Wa"], Mf1[:, D + 2]).reshape(1, D)
    ch = _hdot(p1["Wa"], cf1).reshape(1, D)
    cb1r = (p1["ba"] + p1["bias"]).reshape(1, D)

    y0 = _tc_matmul(x, Mf0[:, :D].T)
    g0 = _sc_gather(y0, src_p)
    mh, mt = _tc_edge(g0, ea_p, ed_p, u0r, v0r, w0r, c0r)
    Ph = _sc_scatter_add(mh, dst_p, NPAD)
    Pt = _sc_scatter_add(mt, dst_p, NPAD)
    z1, R, C = _tc_combine(Ph[:NPAD], Ph[NPAD:], Pt[:NPAD], Pt[NPAD:],
                           wa0t, b1t, uh, vh, wh, ch, cb0r, cb1r)
    Q = _sc_gather_scatter(z1, src_p, dst_p, NPAD)
    out = _tc_final(Q[:NPAD], Q[NPAD:], C, R)
    return out[:N]
